# Initial kernel scaffold; baseline (speedup 1.0000x reference)
import jax
import jax.numpy as jnp
from jax.experimental import pallas as pl


def kernel(inputs):
    raise NotImplementedError("write your pallas kernel here")


# speedup vs baseline: 1.5146x; 1.5146x over previous
import jax
import jax.numpy as jnp
from jax import lax
from jax.experimental import pallas as pl
from jax.experimental.pallas import tpu as pltpu
from jax.experimental.pallas import tpu_sc as plsc

ROWS = 128
N = 32768
L = 16
NC = 2
NS = 16
NW = NC * NS
ROWS_PER_W = ROWS // NW
CHUNK = N // L
NB = 256
HSZ = NB * L
import numpy as np

SIGN = np.int32(-2**31)


def _to_sortable(b):
    return b ^ ((b >> 31) | SIGN)


def _from_sortable(u):
    return u ^ ((~u >> 31) | SIGN)


def _digit(u, shift):
    return (u >> shift) & 0xFF


def _sort_row(src, dst, hist, iota, shift, first, last):
    lane_base = iota * CHUNK
    ones = jnp.ones((L,), jnp.int32)
    zeros = jnp.zeros((L,), jnp.int32)

    def zero_body(i, c):
        hist[pl.ds(i * L, L)] = zeros
        return c

    lax.fori_loop(0, NB, zero_body, 0, unroll=8)

    def count_body(v, c):
        idx = lane_base + v
        b = plsc.bitcast(plsc.load_gather(src, [idx]), jnp.int32)
        u = _to_sortable(b) if first else b
        cidx = (_digit(u, shift) << 4) + iota
        plsc.addupdate_scatter(hist, [cidx], ones)
        return c

    lax.fori_loop(0, CHUNK, count_body, 0, unroll=8)

    def scan_body(d, tot):
        v = hist[pl.ds(d * L, L)]
        excl = plsc.cumsum(v) - v
        hist[pl.ds(d * L, L)] = excl + tot
        return tot + jnp.sum(v)

    lax.fori_loop(0, NB, scan_body, jnp.int32(0), unroll=4)

    def perm_body(v, c):
        idx = lane_base + v
        b = plsc.bitcast(plsc.load_gather(src, [idx]), jnp.int32)
        u = _to_sortable(b) if first else b
        cidx = (_digit(u, shift) << 4) + iota
        pos = plsc.load_gather(hist, [cidx])
        out = _from_sortable(u) if last else u
        plsc.store_scatter(dst, [pos], plsc.bitcast(out, jnp.float32))
        plsc.addupdate_scatter(hist, [cidx], ones)
        return c

    lax.fori_loop(0, CHUNK, perm_body, 0, unroll=8)


def _body(in_hbm, out_hbm, buf_a, buf_b, hist):
    wid = lax.axis_index("c") * NS + lax.axis_index("s")
    iota = lax.broadcasted_iota(jnp.int32, (L,), 0)
    for r in range(ROWS_PER_W):
        row = wid * ROWS_PER_W + r
        pltpu.sync_copy(in_hbm.at[row], buf_a)
        _sort_row(buf_a, buf_b, hist, iota, 0, first=True, last=False)
        _sort_row(buf_b, buf_a, hist, iota, 8, first=False, last=False)
        _sort_row(buf_a, buf_b, hist, iota, 16, first=False, last=False)
        _sort_row(buf_b, buf_a, hist, iota, 24, first=False, last=True)
        pltpu.sync_copy(buf_a, out_hbm.at[row])


@jax.jit
def kernel(inputs):
    mesh = plsc.VectorSubcoreMesh(
        core_axis_name="c", subcore_axis_name="s", num_cores=NC
    )
    run = pl.kernel(
        _body,
        out_type=jax.ShapeDtypeStruct((ROWS, N), jnp.float32),
        mesh=mesh,
        scratch_types=[
            pltpu.VMEM((N,), jnp.float32),
            pltpu.VMEM((N,), jnp.float32),
            pltpu.VMEM((HSZ,), jnp.int32),
        ],
        compiler_params=pltpu.CompilerParams(needs_layout_passes=False),
    )
    return run(inputs)


# speedup vs baseline: 1.7625x; 1.1637x over previous
import jax
import jax.numpy as jnp
from jax import lax
from jax.experimental import pallas as pl
from jax.experimental.pallas import tpu as pltpu
from jax.experimental.pallas import tpu_sc as plsc
import numpy as np

ROWS = 128
N = 32768
L = 16
NC = 2
NS = 16
NW = NC * NS
ROWS_PER_W = ROWS // NW
CHUNK = N // L
S = 8
SUB = CHUNK // S
NB = 256
HGRP = NB * L
HSZ = S * HGRP
SIGN = np.int32(-2**31)


def _to_sortable(b):
    return b ^ ((b >> 31) | SIGN)


def _from_sortable(u):
    return u ^ ((~u >> 31) | SIGN)


def _digit(u, shift):
    return (u >> shift) & 0xFF


def _sort_pass(src, dst, hist, tot, iota, shift, first, last):
    ones = jnp.ones((L,), jnp.int32)
    zeros = jnp.zeros((L,), jnp.int32)
    lane0 = iota == 0
    lane_base = iota * CHUNK

    @plsc.parallel_loop(0, HSZ // L, unroll=8)
    def _(i):
        hist[pl.ds(i * L, L)] = zeros

    @plsc.parallel_loop(0, CHUNK, unroll=8)
    def _(i):
        s = i & (S - 1)
        v = i >> 3
        b = plsc.bitcast(
            plsc.load_gather(src, [lane_base + (s * SUB + v)]), jnp.int32)
        u = _to_sortable(b) if first else b
        cidx = (_digit(u, shift) << 4) + (iota + (s << 12))
        plsc.addupdate_scatter(hist, [cidx], ones)

    @plsc.parallel_loop(0, NB, unroll=2)
    def _(d):
        off = d * L
        vs = [hist[pl.ds(s * HGRP + off, L)] for s in range(S)]
        sumv = ((vs[0] + vs[1]) + (vs[2] + vs[3])) + (
            (vs[4] + vs[5]) + (vs[6] + vs[7]))
        excl = plsc.cumsum(sumv) - sumv
        p = excl
        for s in range(S):
            hist[pl.ds(s * HGRP + off, L)] = p
            p = p + vs[s]
        total = jnp.sum(sumv)
        plsc.store_scatter(tot, [jnp.full((L,), d, jnp.int32)],
                           jnp.full((L,), total, jnp.int32), mask=lane0)

    def scan_b(i, carry):
        v = tot[pl.ds(i * L, L)]
        tot[pl.ds(i * L, L)] = plsc.cumsum(v) - v + carry
        return carry + jnp.sum(v)

    lax.fori_loop(0, NB // L, scan_b, jnp.int32(0))

    @plsc.parallel_loop(0, NB, unroll=4)
    def _(d):
        base = plsc.load_gather(tot, [jnp.full((L,), d, jnp.int32)])
        off = d * L
        for s in range(S):
            plsc.addupdate(hist.at[pl.ds(s * HGRP + off, L)], base)

    def perm_body(i, c):
        s = i & (S - 1)
        v = i >> 3
        b = plsc.bitcast(
            plsc.load_gather(src, [lane_base + (s * SUB + v)]), jnp.int32)
        u = _to_sortable(b) if first else b
        cidx = (_digit(u, shift) << 4) + (iota + (s << 12))
        pos = plsc.load_gather(hist, [cidx])
        out = _from_sortable(u) if last else u
        plsc.store_scatter(dst, [pos], plsc.bitcast(out, jnp.float32))
        plsc.addupdate_scatter(hist, [cidx], ones)
        return c

    lax.fori_loop(0, CHUNK, perm_body, 0, unroll=8)


def _body(in_hbm, out_hbm, buf_a, buf_b, hist, tot):
    wid = lax.axis_index("c") * NS + lax.axis_index("s")
    iota = lax.broadcasted_iota(jnp.int32, (L,), 0)
    for r in range(ROWS_PER_W):
        row = wid * ROWS_PER_W + r
        pltpu.sync_copy(in_hbm.at[row], buf_a)
        _sort_pass(buf_a, buf_b, hist, tot, iota, 0, first=True, last=False)
        _sort_pass(buf_b, buf_a, hist, tot, iota, 8, first=False, last=False)
        _sort_pass(buf_a, buf_b, hist, tot, iota, 16, first=False, last=False)
        _sort_pass(buf_b, buf_a, hist, tot, iota, 24, first=False, last=True)
        pltpu.sync_copy(buf_a, out_hbm.at[row])


@jax.jit
def kernel(inputs):
    mesh = plsc.VectorSubcoreMesh(
        core_axis_name="c", subcore_axis_name="s", num_cores=NC
    )
    run = pl.kernel(
        _body,
        out_type=jax.ShapeDtypeStruct((ROWS, N), jnp.float32),
        mesh=mesh,
        scratch_types=[
            pltpu.VMEM((N,), jnp.float32),
            pltpu.VMEM((N,), jnp.float32),
            pltpu.VMEM((HSZ,), jnp.int32),
            pltpu.VMEM((NB,), jnp.int32),
        ],
        compiler_params=pltpu.CompilerParams(needs_layout_passes=False),
    )
    return run(inputs)
